# baseline (speedup 1.0000x reference)
import functools

import jax
import jax.numpy as jnp
from jax import lax
from jax.experimental import pallas as pl
from jax.experimental.pallas import tpu as pltpu
from jax.experimental.pallas import tpu_sc as plsc

_NC = 2
_NS = 16
_NW = _NC * _NS

_CHUNK = 2048
_UNROLL = 4


@functools.lru_cache(maxsize=None)
def _make_gather_t(nrows: int, v: int, b: int, d: int):
    assert nrows % _NW == 0
    per_w = nrows // _NW
    nchunks = b // _CHUNK
    inner = _CHUNK // (16 * _UNROLL)
    dshift = d.bit_length() - 1
    assert 1 << dshift == d
    ntiles = v // 128
    halfa = (ntiles // 2) * 128
    bmain = v - halfa - (v - ntiles * 128)
    tail = v - ntiles * 128
    halfb = v - halfa
    assert halfa % 128 == 0 and bmain % 128 == 0 and tail < 128

    mesh = plsc.VectorSubcoreMesh(core_axis_name="c", subcore_axis_name="s")

    @functools.partial(
        pl.kernel,
        mesh=mesh,
        out_type=jax.ShapeDtypeStruct((nrows, b), jnp.float32),
        scratch_types=[
            pltpu.VMEM((halfa,), jnp.float32),
            pltpu.VMEM((halfb,), jnp.float32),
            pltpu.VMEM((_CHUNK,), jnp.int32),
            pltpu.VMEM((_CHUNK,), jnp.int32),
            pltpu.VMEM((b,), jnp.float32),
            pltpu.SemaphoreType.DMA,
            pltpu.SemaphoreType.DMA,
            pltpu.SemaphoreType.DMA,
            pltpu.SemaphoreType.DMA,
            pltpu.SemaphoreType.DMA,
        ],
        compiler_params=pltpu.CompilerParams(
            use_tc_tiling_on_sc=True, needs_layout_passes=False
        ),
    )
    def gather_kernel(
        tbl_hbm, idx_hbm, tails_hbm, out_hbm,
        h0, h1, ia, ib, orow, sh0, sh1, sia, sib, sout,
    ):
        sid = lax.axis_index("s")
        cid = lax.axis_index("c")
        wid = sid * _NC + cid
        idx_bufs = (ia, ib)
        sidx = (sia, sib)

        r0 = wid * per_w
        pltpu.async_copy(tbl_hbm.at[r0, :].at[pl.ds(0, halfa)], h0, sh0)

        def row_body(i, c0):
            r = wid * per_w + i
            f = lax.shift_right_logical(r, dshift)
            ibase = f * b

            pltpu.async_copy(
                tbl_hbm.at[r, :].at[pl.ds(halfa, bmain)],
                h1.at[pl.ds(0, bmain)],
                sh1,
            )
            if tail:
                pltpu.async_copy(
                    tails_hbm.at[pl.ds(r * tail, tail)],
                    h1.at[pl.ds(bmain, tail)],
                    sh1,
                )
            pltpu.async_copy(idx_hbm.at[pl.ds(ibase, _CHUNK)], ia, sia)
            pltpu.async_copy(
                idx_hbm.at[pl.ds(ibase + _CHUNK, _CHUNK)], ib, sib
            )

            @pl.when(i > 0)
            def _():
                pltpu.make_async_copy(orow, out_hbm.at[r, :], sout).wait()

            pltpu.make_async_copy(
                tbl_hbm.at[r, :].at[pl.ds(0, halfa)], h0, sh0
            ).wait()

            for c in range(nchunks):
                u = c % 2
                pltpu.make_async_copy(
                    idx_hbm.at[pl.ds(ibase + c * _CHUNK, _CHUNK)],
                    idx_bufs[u],
                    sidx[u],
                ).wait()

                def vec_a(t, c2, _iv=idx_bufs[u], _off=c * _CHUNK):
                    base = t * (16 * _UNROLL)
                    for uu in range(_UNROLL):
                        ii = _iv[pl.ds(base + uu * 16, 16)]
                        cl = jnp.minimum(ii, halfa - 1)
                        orow[pl.ds(_off + base + uu * 16, 16)] = (
                            plsc.load_gather(h0, [cl])
                        )
                    return c2

                lax.fori_loop(0, inner, vec_a, 0)
                if c + 2 < nchunks:
                    pltpu.async_copy(
                        idx_hbm.at[pl.ds(ibase + (c + 2) * _CHUNK, _CHUNK)],
                        idx_bufs[u],
                        sidx[u],
                    )

            pltpu.async_copy(idx_hbm.at[pl.ds(ibase, _CHUNK)], ia, sia)
            pltpu.async_copy(
                idx_hbm.at[pl.ds(ibase + _CHUNK, _CHUNK)], ib, sib
            )

            pltpu.make_async_copy(
                tbl_hbm.at[r, :].at[pl.ds(halfa, bmain)],
                h1.at[pl.ds(0, bmain)],
                sh1,
            ).wait()
            if tail:
                pltpu.make_async_copy(
                    tails_hbm.at[pl.ds(r * tail, tail)],
                    h1.at[pl.ds(bmain, tail)],
                    sh1,
                ).wait()

            @pl.when(i + 1 < per_w)
            def _():
                pltpu.async_copy(
                    tbl_hbm.at[r + 1, :].at[pl.ds(0, halfa)], h0, sh0
                )

            for c in range(nchunks):
                u = c % 2
                pltpu.make_async_copy(
                    idx_hbm.at[pl.ds(ibase + c * _CHUNK, _CHUNK)],
                    idx_bufs[u],
                    sidx[u],
                ).wait()

                def vec_b(t, c2, _iv=idx_bufs[u], _off=c * _CHUNK):
                    base = t * (16 * _UNROLL)
                    for uu in range(_UNROLL):
                        ii = _iv[pl.ds(base + uu * 16, 16)]
                        local = ii - halfa
                        cl = jnp.maximum(local, 0)
                        g = plsc.load_gather(h1, [cl])
                        prev = orow[pl.ds(_off + base + uu * 16, 16)]
                        orow[pl.ds(_off + base + uu * 16, 16)] = jnp.where(
                            local >= 0, g, prev
                        )
                    return c2

                lax.fori_loop(0, inner, vec_b, 0)
                if c + 2 < nchunks:
                    pltpu.async_copy(
                        idx_hbm.at[pl.ds(ibase + (c + 2) * _CHUNK, _CHUNK)],
                        idx_bufs[u],
                        sidx[u],
                    )

            pltpu.async_copy(orow, out_hbm.at[r, :], sout)
            return c0

        lax.fori_loop(0, per_w, row_body, 0)
        pltpu.make_async_copy(orow, out_hbm.at[0, :], sout).wait()

    return gather_kernel


def _mlp_t_body(xt_ref, we_ref, be_ref, wd_ref, bd_ref, o_ref):
    xt = xt_ref[...]
    z = lax.dot_general(
        we_ref[...], xt, (((0,), (0,)), ((), ())),
        preferred_element_type=jnp.float32,
    )
    z = jnp.maximum(z + be_ref[...], 0.0)
    y = lax.dot_general(
        wd_ref[...], z, (((0,), (0,)), ((), ())),
        preferred_element_type=jnp.float32,
    )
    y = y + bd_ref[...]
    o_ref[...] = 1.0 / (1.0 + jnp.exp(-y))


@functools.lru_cache(maxsize=None)
def _make_mlp_t(b: int, out_dim: int, latent: int, bn: int):
    grid = (b // bn,)
    return pl.pallas_call(
        _mlp_t_body,
        grid=grid,
        in_specs=[
            pl.BlockSpec((out_dim, bn), lambda i: (0, i)),
            pl.BlockSpec((out_dim, latent), lambda i: (0, 0)),
            pl.BlockSpec((latent, 1), lambda i: (0, 0)),
            pl.BlockSpec((latent, out_dim), lambda i: (0, 0)),
            pl.BlockSpec((out_dim, 1), lambda i: (0, 0)),
        ],
        out_specs=pl.BlockSpec((out_dim, bn), lambda i: (0, i)),
        out_shape=jax.ShapeDtypeStruct((out_dim, b), jnp.float32),
    )


def kernel(indices, tables, W_enc, b_enc, W_dec, b_dec):
    b, f = indices.shape
    _, v, d = tables.shape
    out_dim, latent = W_enc.shape

    planes = tables.transpose(0, 2, 1).reshape(f * d, v)
    idx_flat = indices.astype(jnp.int32).T.reshape(-1)

    tails = lax.slice(planes, (0, (v // 128) * 128), (f * d, v)).reshape(-1)
    xt = _make_gather_t(f * d, v, b, d)(planes, idx_flat, tails)

    mlp = _make_mlp_t(b, out_dim, latent, 2048)
    out_t = mlp(
        xt, W_enc, b_enc.reshape(latent, 1), W_dec, b_dec.reshape(out_dim, 1)
    )
    return out_t.T


# speedup vs baseline: 1.5767x; 1.5767x over previous
import functools

import jax
import jax.numpy as jnp
from jax import lax
from jax.experimental import pallas as pl
from jax.experimental.pallas import tpu as pltpu
from jax.experimental.pallas import tpu_sc as plsc

_NC = 2
_NS = 16
_NW = _NC * _NS

_CHUNK = 2048
_UNROLL = 4


@functools.lru_cache(maxsize=None)
def _make_gather_t(nrows: int, v: int, b: int, d: int):
    assert nrows % _NW == 0
    per_w = nrows // _NW
    nchunks = b // _CHUNK
    inner = _CHUNK // (16 * _UNROLL)
    dshift = d.bit_length() - 1
    assert 1 << dshift == d

    mesh = plsc.VectorSubcoreMesh(core_axis_name="c", subcore_axis_name="s")

    @functools.partial(
        pl.kernel,
        mesh=mesh,
        out_type=jax.ShapeDtypeStruct((nrows, b), jnp.float32),
        scratch_types=[
            pltpu.VMEM((v,), jnp.float32),
            pltpu.VMEM((b,), jnp.int32),
            pltpu.VMEM((_CHUNK,), jnp.float32),
            pltpu.VMEM((_CHUNK,), jnp.float32),
            pltpu.SemaphoreType.DMA,
            pltpu.SemaphoreType.DMA,
            pltpu.SemaphoreType.DMA,
        ],
        compiler_params=pltpu.CompilerParams(
            use_tc_tiling_on_sc=True, needs_layout_passes=False
        ),
    )
    def gather_kernel(
        tbl_hbm, idx_hbm, out_hbm,
        plane_v, idx_v, out_a, out_b, si, so_a, so_b,
    ):
        wid = lax.axis_index("s") * _NC + lax.axis_index("c")
        out_bufs = (out_a, out_b)
        so = (so_a, so_b)

        def row_body(i, c0):
            r = wid * per_w + i
            f = lax.shift_right_logical(r, dshift)
            ibase = f * b
            pltpu.async_copy(idx_hbm.at[pl.ds(ibase, b)], idx_v, si)
            pltpu.sync_copy(tbl_hbm.at[r, :], plane_v)
            pltpu.make_async_copy(
                idx_hbm.at[pl.ds(ibase, b)], idx_v, si
            ).wait()

            for c in range(nchunks):
                u = c % 2
                if c >= 2:
                    pltpu.make_async_copy(
                        out_bufs[u],
                        out_hbm.at[r, pl.ds((c - 2) * _CHUNK, _CHUNK)],
                        so[u],
                    ).wait()
                def vec_body(t, c2, _ov=out_bufs[u], _coff=c * _CHUNK):
                    base = t * (16 * _UNROLL)
                    for uu in range(_UNROLL):
                        ii = idx_v[pl.ds(_coff + base + uu * 16, 16)]
                        _ov[pl.ds(base + uu * 16, 16)] = plsc.load_gather(
                            plane_v, [ii]
                        )
                    return c2

                lax.fori_loop(0, inner, vec_body, 0)
                pltpu.async_copy(
                    out_bufs[u], out_hbm.at[r, pl.ds(c * _CHUNK, _CHUNK)], so[u]
                )

            for c in (nchunks - 2, nchunks - 1):
                u = c % 2
                pltpu.make_async_copy(
                    out_bufs[u], out_hbm.at[r, pl.ds(c * _CHUNK, _CHUNK)], so[u]
                ).wait()
            return c0

        lax.fori_loop(0, per_w, row_body, 0)

    return gather_kernel


def _mlp_t_body(xt_ref, we_ref, be_ref, wd_ref, bd_ref, o_ref):
    xt = xt_ref[...]
    z = lax.dot_general(
        we_ref[...], xt, (((0,), (0,)), ((), ())),
        preferred_element_type=jnp.float32,
    )
    z = jnp.maximum(z + be_ref[...], 0.0)
    y = lax.dot_general(
        wd_ref[...], z, (((0,), (0,)), ((), ())),
        preferred_element_type=jnp.float32,
    )
    y = y + bd_ref[...]
    o_ref[...] = 1.0 / (1.0 + jnp.exp(-y))


@functools.lru_cache(maxsize=None)
def _make_mlp_t(b: int, out_dim: int, latent: int, bn: int):
    grid = (b // bn,)
    return pl.pallas_call(
        _mlp_t_body,
        grid=grid,
        in_specs=[
            pl.BlockSpec((out_dim, bn), lambda i: (0, i)),
            pl.BlockSpec((out_dim, latent), lambda i: (0, 0)),
            pl.BlockSpec((latent, 1), lambda i: (0, 0)),
            pl.BlockSpec((latent, out_dim), lambda i: (0, 0)),
            pl.BlockSpec((out_dim, 1), lambda i: (0, 0)),
        ],
        out_specs=pl.BlockSpec((out_dim, bn), lambda i: (0, i)),
        out_shape=jax.ShapeDtypeStruct((out_dim, b), jnp.float32),
    )


def kernel(indices, tables, W_enc, b_enc, W_dec, b_dec):
    b, f = indices.shape
    _, v, d = tables.shape
    out_dim, latent = W_enc.shape

    planes = tables.transpose(0, 2, 1).reshape(f * d, v)
    idx_flat = indices.astype(jnp.int32).T.reshape(-1)

    xt = _make_gather_t(f * d, v, b, d)(planes, idx_flat)

    mlp = _make_mlp_t(b, out_dim, latent, 2048)
    out_t = mlp(
        xt, W_enc, b_enc.reshape(latent, 1), W_dec, b_dec.reshape(out_dim, 1)
    )
    return out_t.T


# speedup vs baseline: 2.1084x; 1.3373x over previous
import functools

import jax
import jax.numpy as jnp
from jax import lax
from jax.experimental import pallas as pl
from jax.experimental.pallas import tpu as pltpu
from jax.experimental.pallas import tpu_sc as plsc

_NC = 2
_NS = 16
_NW = _NC * _NS

_CHUNK = 2048
_UNROLL = 8


@functools.lru_cache(maxsize=None)
def _make_gather_t(nrows: int, v: int, b: int, d: int):
    assert nrows % _NW == 0
    per_w = nrows // _NW
    nchunks = b // _CHUNK
    inner = _CHUNK // (16 * _UNROLL)
    dshift = d.bit_length() - 1
    assert 1 << dshift == d

    mesh = plsc.VectorSubcoreMesh(core_axis_name="c", subcore_axis_name="s")

    @functools.partial(
        pl.kernel,
        mesh=mesh,
        out_type=jax.ShapeDtypeStruct((nrows, b), jnp.float32),
        scratch_types=[
            pltpu.VMEM((v,), jnp.float32),
            pltpu.VMEM((_CHUNK,), jnp.int32),
            pltpu.VMEM((_CHUNK,), jnp.int32),
            pltpu.VMEM((_CHUNK,), jnp.float32),
            pltpu.VMEM((_CHUNK,), jnp.float32),
            pltpu.SemaphoreType.DMA,
            pltpu.SemaphoreType.DMA,
            pltpu.SemaphoreType.DMA,
            pltpu.SemaphoreType.DMA,
        ],
        compiler_params=pltpu.CompilerParams(
            use_tc_tiling_on_sc=True, needs_layout_passes=False
        ),
    )
    def gather_kernel(
        tbl_hbm, idx_hbm, out_hbm,
        plane_v, idx_a, idx_b, out_a, out_b, si_a, si_b, so_a, so_b,
    ):
        wid = lax.axis_index("s") * _NC + lax.axis_index("c")
        idx_bufs = (idx_a, idx_b)
        out_bufs = (out_a, out_b)
        si = (si_a, si_b)
        so = (so_a, so_b)

        def row_body(i, c0):
            r = wid * per_w + i
            f = lax.shift_right_logical(r, dshift)
            ibase = f * b
            pltpu.async_copy(
                idx_hbm.at[pl.ds(ibase, _CHUNK)], idx_a, si_a
            )
            pltpu.async_copy(
                idx_hbm.at[pl.ds(ibase + _CHUNK, _CHUNK)], idx_b, si_b
            )
            pltpu.sync_copy(tbl_hbm.at[r, :], plane_v)

            for c in range(nchunks):
                u = c % 2
                if c >= 2:
                    pltpu.make_async_copy(
                        out_bufs[u],
                        out_hbm.at[r, pl.ds((c - 2) * _CHUNK, _CHUNK)],
                        so[u],
                    ).wait()
                pltpu.make_async_copy(
                    idx_hbm.at[pl.ds(ibase + c * _CHUNK, _CHUNK)],
                    idx_bufs[u],
                    si[u],
                ).wait()

                def vec_body(t, c2, _iv=idx_bufs[u], _ov=out_bufs[u]):
                    base = t * (16 * _UNROLL)
                    for uu in range(_UNROLL):
                        ii = _iv[pl.ds(base + uu * 16, 16)]
                        _ov[pl.ds(base + uu * 16, 16)] = plsc.load_gather(
                            plane_v, [ii]
                        )
                    return c2

                lax.fori_loop(0, inner, vec_body, 0)
                pltpu.async_copy(
                    out_bufs[u], out_hbm.at[r, pl.ds(c * _CHUNK, _CHUNK)], so[u]
                )
                if c + 2 < nchunks:
                    pltpu.async_copy(
                        idx_hbm.at[pl.ds(ibase + (c + 2) * _CHUNK, _CHUNK)],
                        idx_bufs[u],
                        si[u],
                    )

            for c in (nchunks - 2, nchunks - 1):
                u = c % 2
                pltpu.make_async_copy(
                    out_bufs[u], out_hbm.at[r, pl.ds(c * _CHUNK, _CHUNK)], so[u]
                ).wait()
            return c0

        lax.fori_loop(0, per_w, row_body, 0)

    return gather_kernel


def _mlp_t_body(xt_ref, we_ref, be_ref, wd_ref, bd_ref, o_ref):
    xt = xt_ref[...]
    z = lax.dot_general(
        we_ref[...], xt, (((0,), (0,)), ((), ())),
        preferred_element_type=jnp.float32,
    )
    z = jnp.maximum(z + be_ref[...], 0.0)
    y = lax.dot_general(
        wd_ref[...], z, (((0,), (0,)), ((), ())),
        preferred_element_type=jnp.float32,
    )
    y = y + bd_ref[...]
    o_ref[...] = 1.0 / (1.0 + jnp.exp(-y))


@functools.lru_cache(maxsize=None)
def _make_mlp_t(b: int, out_dim: int, latent: int, bn: int):
    grid = (b // bn,)
    return pl.pallas_call(
        _mlp_t_body,
        grid=grid,
        in_specs=[
            pl.BlockSpec((out_dim, bn), lambda i: (0, i)),
            pl.BlockSpec((out_dim, latent), lambda i: (0, 0)),
            pl.BlockSpec((latent, 1), lambda i: (0, 0)),
            pl.BlockSpec((latent, out_dim), lambda i: (0, 0)),
            pl.BlockSpec((out_dim, 1), lambda i: (0, 0)),
        ],
        out_specs=pl.BlockSpec((out_dim, bn), lambda i: (0, i)),
        out_shape=jax.ShapeDtypeStruct((out_dim, b), jnp.float32),
    )


def kernel(indices, tables, W_enc, b_enc, W_dec, b_dec):
    b, f = indices.shape
    _, v, d = tables.shape
    out_dim, latent = W_enc.shape

    planes = tables.transpose(0, 2, 1).reshape(f * d, v)
    idx_flat = indices.astype(jnp.int32).T.reshape(-1)

    xt = _make_gather_t(f * d, v, b, d)(planes, idx_flat)

    mlp = _make_mlp_t(b, out_dim, latent, 2048)
    out_t = mlp(
        xt, W_enc, b_enc.reshape(latent, 1), W_dec, b_dec.reshape(out_dim, 1)
    )
    return out_t.T
